# baseline (speedup 1.0000x reference)
import functools

import jax
import jax.numpy as jnp
from jax import lax
from jax.experimental import pallas as pl
from jax.experimental.pallas import tpu as pltpu
from jax.experimental.pallas import tpu_sc as plsc

N = 10000
E = 320000
NG = 64
D_IN = 128

NC, NS = 2, 16
NW = NC * NS
N_PAD = 10240
RPS = N_PAD // NS
E_PAD = 327680
EPW = E_PAD // NW
CHUNK = 128
NCHUNK = EPW // CHUNK

BLK = 1024
NBLK = N_PAD // BLK

D1 = 128
D2 = 160
D3 = 128


@functools.cache
def _sc_kernels():
    mesh = plsc.VectorSubcoreMesh(
        core_axis_name="c", subcore_axis_name="s",
        num_cores=NC, num_subcores=NS)

    CDEG = 1024
    NCDEG = EPW // CDEG

    @functools.partial(
        pl.kernel,
        mesh=mesh,
        out_type=jax.ShapeDtypeStruct((NC, N_PAD), jnp.float32),
        scratch_types=[
            pltpu.VMEM((CDEG,), jnp.int32),
            pltpu.VMEM((CDEG,), jnp.int32),
            pltpu.VMEM((CDEG,), jnp.float32),
            pltpu.VMEM((RPS,), jnp.float32),
            pltpu.VMEM_SHARED((N_PAD,), jnp.float32),
            pltpu.SemaphoreType.DMA,
            pltpu.SemaphoreType.DMA,
            pltpu.SemaphoreType.DMA,
            pltpu.SemaphoreType.DMA,
        ],
        name="gcn_deg",
    )
    def deg(dst_hbm, out_hbm, dst_va, dst_vb, ones_v, zbuf_v, acc_sh,
            d0, d1, s0, s1):
        dbuf = (dst_va, dst_vb)
        dsem = (d0, d1)
        ssem = (s0, s1)
        c = lax.axis_index("c")
        s = lax.axis_index("s")
        wid = c * NS + s

        def fill(i, _):
            ones_v[pl.ds(i * 16, 16)] = jnp.full((16,), 1.0, jnp.float32)
            return 0
        lax.fori_loop(0, CDEG // 16, fill, 0)

        def zero(i, _):
            zbuf_v[pl.ds(i * 16, 16)] = jnp.zeros((16,), jnp.float32)
            return 0
        lax.fori_loop(0, RPS // 16, zero, 0)

        r0 = s * RPS
        pltpu.sync_copy(zbuf_v, acc_sh.at[pl.ds(r0, RPS)])
        plsc.subcore_barrier()

        e0 = wid * EPW
        for p in range(2):
            pltpu.async_copy(dst_hbm.at[pl.ds(e0 + p * CDEG, CDEG)],
                             dbuf[p], dsem[p])

        def body(k, _):
            for p in range(2):
                i = 2 * k + p
                pltpu.make_async_copy(
                    dst_hbm.at[pl.ds(e0 + i * CDEG, CDEG)],
                    dbuf[p], dsem[p]).wait()
                pltpu.async_copy(ones_v, acc_sh.at[dbuf[p]], ssem[p],
                                 add=True)

                @pl.when(i + 2 < NCDEG)
                def _():
                    pltpu.make_async_copy(
                        ones_v, acc_sh.at[dbuf[p]], ssem[p]).wait()
                    pltpu.async_copy(
                        dst_hbm.at[pl.ds(e0 + (i + 2) * CDEG, CDEG)],
                        dbuf[p], dsem[p])
            return 0
        lax.fori_loop(0, NCDEG // 2, body, 0)

        for p in range(2):
            pltpu.make_async_copy(
                ones_v, acc_sh.at[dbuf[p]], ssem[p]).wait()
        plsc.subcore_barrier()
        pltpu.sync_copy(acc_sh.at[pl.ds(r0, RPS)],
                        out_hbm.at[c, pl.ds(r0, RPS)])

    def make_prop(d, tag):
        CH = 160
        NCH = EPW // CH

        @functools.partial(
            pl.kernel,
            mesh=mesh,
            out_type=jax.ShapeDtypeStruct((NC, N_PAD, d), jnp.float32),
            scratch_types=(
                [pltpu.VMEM((CH,), jnp.int32)] * 4
                + [pltpu.VMEM((CH, d), jnp.float32)] * 2
                + [pltpu.VMEM_SHARED((N_PAD, d), jnp.float32)]
                + [pltpu.SemaphoreType.DMA] * 8
            ),
            name=f"gcn_prop_{tag}",
        )
        def prop(t_hbm, src_hbm, dst_hbm, zr_hbm, out_hbm, *refs):
            sbuf = refs[0:2]
            dbuf = refs[2:4]
            rbuf = refs[4:6]
            acc_sh = refs[6]
            sems = refs[7:]
            gsem = sems[0:2]
            ssem = sems[2:4]
            isem = sems[4:6]
            dsem = sems[6:8]
            c = lax.axis_index("c")
            s = lax.axis_index("s")
            wid = c * NS + s

            r0 = s * RPS
            for k in range(RPS // CHUNK):
                pltpu.sync_copy(zr_hbm, acc_sh.at[pl.ds(r0 + k * CHUNK,
                                                        CHUNK)])
            plsc.subcore_barrier()

            e0 = wid * EPW

            pltpu.sync_copy(src_hbm.at[pl.ds(e0, CH)], sbuf[0])
            pltpu.async_copy(src_hbm.at[pl.ds(e0 + CH, CH)], sbuf[1],
                             isem[1])
            pltpu.async_copy(dst_hbm.at[pl.ds(e0, CH)], dbuf[0], dsem[0])
            pltpu.async_copy(t_hbm.at[sbuf[0]], rbuf[0], gsem[0])

            def body(k, _):
                for p in range(2):
                    i = 2 * k + p
                    o = 1 - p
                    pltpu.make_async_copy(
                        t_hbm.at[sbuf[p]], rbuf[p], gsem[p]).wait()

                    @pl.when(i + 2 < NCH)
                    def _():
                        pltpu.async_copy(
                            src_hbm.at[pl.ds(e0 + (i + 2) * CH, CH)],
                            sbuf[p], isem[p])

                    pltpu.make_async_copy(
                        dst_hbm.at[pl.ds(e0 + i * CH, CH)],
                        dbuf[p], dsem[p]).wait()
                    pltpu.async_copy(rbuf[p], acc_sh.at[dbuf[p]],
                                     ssem[p], add=True)

                    @pl.when(jnp.logical_and(i >= 1, i + 1 < NCH))
                    def _():
                        pltpu.make_async_copy(
                            rbuf[o], acc_sh.at[dbuf[o]], ssem[o]).wait()

                    @pl.when(i + 1 < NCH)
                    def _():
                        j = i + 1
                        pltpu.make_async_copy(
                            src_hbm.at[pl.ds(e0 + j * CH, CH)],
                            sbuf[o], isem[o]).wait()
                        pltpu.async_copy(t_hbm.at[sbuf[o]], rbuf[o],
                                         gsem[o])
                        pltpu.async_copy(dst_hbm.at[pl.ds(e0 + j * CH, CH)],
                                         dbuf[o], dsem[o])
                return 0
            lax.fori_loop(0, NCH // 2, body, 0)

            for p in range(2):
                pltpu.make_async_copy(
                    rbuf[p], acc_sh.at[dbuf[p]], ssem[p]).wait()
            plsc.subcore_barrier()
            pltpu.sync_copy(acc_sh.at[pl.ds(r0, RPS)],
                            out_hbm.at[c, pl.ds(r0, RPS)])

        return prop

    return deg, make_prop(D1, "p1"), make_prop(D3, "p3")


def _tcA(deg3, x_p, w1p):
    def body(deg_ref, x_ref, w_ref, dis_ref, t_ref):
        deg = deg_ref[0] + deg_ref[1] + 1.0
        dis = lax.rsqrt(deg)
        dis_ref[...] = dis
        t_ref[...] = jnp.dot(x_ref[...], w_ref[...],
                             preferred_element_type=jnp.float32) * dis
    return pl.pallas_call(
        body,
        grid=(NBLK,),
        in_specs=[
            pl.BlockSpec((NC, BLK, 1), lambda i: (0, i, 0)),
            pl.BlockSpec((BLK, D_IN), lambda i: (i, 0)),
            pl.BlockSpec((D_IN, D1), lambda i: (0, 0)),
        ],
        out_specs=[
            pl.BlockSpec((BLK, 1), lambda i: (i, 0)),
            pl.BlockSpec((BLK, D1), lambda i: (i, 0)),
        ],
        out_shape=[
            jax.ShapeDtypeStruct((N_PAD, 1), jnp.float32),
            jax.ShapeDtypeStruct((N_PAD, D1), jnp.float32),
        ],
    )(deg3, x_p, w1p)


def _tcB(p1, t1, dis, b1p):
    def body(p_ref, t_ref, dis_ref, b_ref, u_ref):
        dis = dis_ref[...]
        h = dis * (p_ref[0] + p_ref[1] + t_ref[...]) + b_ref[...]
        u_ref[...] = jnp.maximum(h, 0.0) * dis
    return pl.pallas_call(
        body,
        grid=(NBLK,),
        in_specs=[
            pl.BlockSpec((NC, BLK, D1), lambda i: (0, i, 0)),
            pl.BlockSpec((BLK, D1), lambda i: (i, 0)),
            pl.BlockSpec((BLK, 1), lambda i: (i, 0)),
            pl.BlockSpec((1, D1), lambda i: (0, 0)),
        ],
        out_specs=pl.BlockSpec((BLK, D1), lambda i: (i, 0)),
        out_shape=jax.ShapeDtypeStruct((N_PAD, D1), jnp.float32),
    )(p1, t1, dis, b1p)


def _tcC(p2, u2, dis, w2p, b2p, w3p):
    def body(p_ref, u_ref, dis_ref, w2_ref, b2_ref, w3_ref, t3_ref):
        dis = dis_ref[...]
        ah = dis * (p_ref[0] + p_ref[1] + u_ref[...])
        h2 = jnp.maximum(
            jnp.dot(ah, w2_ref[...], preferred_element_type=jnp.float32)
            + b2_ref[...], 0.0)
        t3_ref[...] = jnp.dot(h2, w3_ref[...],
                              preferred_element_type=jnp.float32) * dis
    return pl.pallas_call(
        body,
        grid=(NBLK,),
        in_specs=[
            pl.BlockSpec((NC, BLK, D1), lambda i: (0, i, 0)),
            pl.BlockSpec((BLK, D1), lambda i: (i, 0)),
            pl.BlockSpec((BLK, 1), lambda i: (i, 0)),
            pl.BlockSpec((D1, D2), lambda i: (0, 0)),
            pl.BlockSpec((1, D2), lambda i: (0, 0)),
            pl.BlockSpec((D2, D3), lambda i: (0, 0)),
        ],
        out_specs=pl.BlockSpec((BLK, D3), lambda i: (i, 0)),
        out_shape=jax.ShapeDtypeStruct((N_PAD, D3), jnp.float32),
    )(p2, u2, dis, w2p, b2p, w3p)


def _tcD(batch_p, p3, t3, dis, b3p, bcol, wop, bo2):
    def body(batch_smem, p_ref, t_ref, dis_ref, b_ref, bcol_ref, wo_ref,
             bo_ref, pool_ref, out_ref):
        i = pl.program_id(0)

        @pl.when(i == 0)
        def _init():
            pool_ref[...] = jnp.full((NG, D3), -jnp.inf, jnp.float32)

        dis = dis_ref[...]
        h3 = jnp.maximum(
            dis * (p_ref[0] + p_ref[1] + t_ref[...]) + b_ref[...], 0.0)
        bcol = bcol_ref[...]
        g0 = batch_smem[i * BLK]
        g1 = jnp.minimum(batch_smem[i * BLK + BLK - 1], NG - 1)
        rowg = lax.broadcasted_iota(jnp.int32, (NG, D3), 0)

        def gbody(g, _):
            m = bcol == g
            v = jnp.where(m, h3, -jnp.inf)
            mx = jnp.max(v, axis=0, keepdims=True)
            upd = jnp.where(rowg == g, jnp.broadcast_to(mx, (NG, D3)),
                            -jnp.inf)
            pool_ref[...] = jnp.maximum(pool_ref[...], upd)
            return 0
        lax.fori_loop(g0, g1 + 1, gbody, 0)

        @pl.when(i == NBLK - 1)
        def _fin():
            z = jnp.dot(pool_ref[...], wo_ref[...],
                        preferred_element_type=jnp.float32) + bo_ref[...]
            out_ref[...] = jax.nn.sigmoid(z)

    pool, out = pl.pallas_call(
        body,
        grid=(NBLK,),
        in_specs=[
            pl.BlockSpec(memory_space=pltpu.SMEM),
            pl.BlockSpec((NC, BLK, D3), lambda i: (0, i, 0)),
            pl.BlockSpec((BLK, D3), lambda i: (i, 0)),
            pl.BlockSpec((BLK, 1), lambda i: (i, 0)),
            pl.BlockSpec((1, D3), lambda i: (0, 0)),
            pl.BlockSpec((BLK, 1), lambda i: (i, 0)),
            pl.BlockSpec((D3, 1), lambda i: (0, 0)),
            pl.BlockSpec((1, 1), lambda i: (0, 0)),
        ],
        out_specs=[
            pl.BlockSpec((NG, D3), lambda i: (0, 0)),
            pl.BlockSpec((NG, 1), lambda i: (0, 0)),
        ],
        out_shape=[
            jax.ShapeDtypeStruct((NG, D3), jnp.float32),
            jax.ShapeDtypeStruct((NG, 1), jnp.float32),
        ],
    )(batch_p, p3, t3, dis, b3p, bcol, wop, bo2)
    return out


def kernel(x, edge_index, batch, W1, b1, W2, b2, W3, b3, Wo, bo):
    src = edge_index[0]
    dst = edge_index[1]
    npad = E_PAD - E
    pad_ids = (N + (jnp.arange(npad, dtype=jnp.int32) % (N_PAD - N))
               ).astype(jnp.int32)
    src_p = jnp.concatenate([src, pad_ids])
    dst_p = jnp.concatenate([dst, pad_ids])

    x_p = jnp.pad(x, ((0, N_PAD - N), (0, 0)))
    w1p = jnp.pad(W1, ((0, 0), (0, D1 - 75)))
    b1p = jnp.pad(b1, (0, D1 - 75))[None, :]
    w2p = jnp.pad(W2, ((0, D1 - 75), (0, D2 - 150)))
    b2p = jnp.pad(b2, (0, D2 - 150))[None, :]
    w3p = jnp.pad(W3, ((0, D2 - 150), (0, D3 - 50)))
    b3p = jnp.pad(b3, (0, D3 - 50))[None, :]
    wop = jnp.pad(Wo, ((0, D3 - 50), (0, 0)))
    bo2 = bo[None, :]
    batch_p = jnp.pad(batch, (0, N_PAD - N), constant_values=NG)
    bcol = batch_p[:, None]

    _deg, _prop1, _prop3 = _sc_kernels()

    zr1 = jnp.zeros((CHUNK, D1), jnp.float32)
    zr3 = jnp.zeros((CHUNK, D3), jnp.float32)

    degp = _deg(dst_p)
    deg3 = degp[:, :, None]
    dis, t1 = _tcA(deg3, x_p, w1p)
    p1 = _prop1(t1, src_p, dst_p, zr1)
    u2 = _tcB(p1, t1, dis, b1p)
    p2 = _prop1(u2, src_p, dst_p, zr1)
    t3 = _tcC(p2, u2, dis, w2p, b2p, w3p)
    p3 = _prop3(t3, src_p, dst_p, zr3)
    return _tcD(batch_p, p3, t3, dis, b3p, bcol, wop, bo2)


# speedup vs baseline: 1.1042x; 1.1042x over previous
import functools

import jax
import jax.numpy as jnp
from jax import lax
from jax.experimental import pallas as pl
from jax.experimental.pallas import tpu as pltpu
from jax.experimental.pallas import tpu_sc as plsc

N = 10000
E = 320000
NG = 64
D_IN = 128

NC, NS = 2, 16
NW = NC * NS
N_PAD = 10240
RPS = N_PAD // NS
E_PAD = 327680
EPW = E_PAD // NW
CHUNK = 128
NCHUNK = EPW // CHUNK

BLK = 1024
NBLK = N_PAD // BLK

D1 = 80
D2 = 160
D3 = 64


@functools.cache
def _sc_kernels():
    mesh = plsc.VectorSubcoreMesh(
        core_axis_name="c", subcore_axis_name="s",
        num_cores=NC, num_subcores=NS)

    CDEG = 1024
    NCDEG = EPW // CDEG

    @functools.partial(
        pl.kernel,
        mesh=mesh,
        out_type=jax.ShapeDtypeStruct((NC, N_PAD), jnp.float32),
        scratch_types=[
            pltpu.VMEM((CDEG,), jnp.int32),
            pltpu.VMEM((CDEG,), jnp.int32),
            pltpu.VMEM((CDEG,), jnp.float32),
            pltpu.VMEM((RPS,), jnp.float32),
            pltpu.VMEM_SHARED((N_PAD,), jnp.float32),
            pltpu.SemaphoreType.DMA,
            pltpu.SemaphoreType.DMA,
            pltpu.SemaphoreType.DMA,
            pltpu.SemaphoreType.DMA,
        ],
        name="gcn_deg",
    )
    def deg(dst_hbm, out_hbm, dst_va, dst_vb, ones_v, zbuf_v, acc_sh,
            d0, d1, s0, s1):
        dbuf = (dst_va, dst_vb)
        dsem = (d0, d1)
        ssem = (s0, s1)
        c = lax.axis_index("c")
        s = lax.axis_index("s")
        wid = c * NS + s

        def fill(i, _):
            ones_v[pl.ds(i * 16, 16)] = jnp.full((16,), 1.0, jnp.float32)
            return 0
        lax.fori_loop(0, CDEG // 16, fill, 0)

        def zero(i, _):
            zbuf_v[pl.ds(i * 16, 16)] = jnp.zeros((16,), jnp.float32)
            return 0
        lax.fori_loop(0, RPS // 16, zero, 0)

        r0 = s * RPS
        pltpu.sync_copy(zbuf_v, acc_sh.at[pl.ds(r0, RPS)])
        plsc.subcore_barrier()

        e0 = wid * EPW
        for p in range(2):
            pltpu.async_copy(dst_hbm.at[pl.ds(e0 + p * CDEG, CDEG)],
                             dbuf[p], dsem[p])

        def body(k, _):
            for p in range(2):
                i = 2 * k + p
                pltpu.make_async_copy(
                    dst_hbm.at[pl.ds(e0 + i * CDEG, CDEG)],
                    dbuf[p], dsem[p]).wait()
                pltpu.async_copy(ones_v, acc_sh.at[dbuf[p]], ssem[p],
                                 add=True)

                @pl.when(i + 2 < NCDEG)
                def _():
                    pltpu.make_async_copy(
                        ones_v, acc_sh.at[dbuf[p]], ssem[p]).wait()
                    pltpu.async_copy(
                        dst_hbm.at[pl.ds(e0 + (i + 2) * CDEG, CDEG)],
                        dbuf[p], dsem[p])
            return 0
        lax.fori_loop(0, NCDEG // 2, body, 0)

        for p in range(2):
            pltpu.make_async_copy(
                ones_v, acc_sh.at[dbuf[p]], ssem[p]).wait()
        plsc.subcore_barrier()
        pltpu.sync_copy(acc_sh.at[pl.ds(r0, RPS)],
                        out_hbm.at[c, pl.ds(r0, RPS)])

    def make_prop(d, tag):
        CH = 160
        NCH = EPW // CH

        @functools.partial(
            pl.kernel,
            mesh=mesh,
            out_type=jax.ShapeDtypeStruct((NC, N_PAD, d), jnp.float32),
            scratch_types=(
                [pltpu.VMEM((CH,), jnp.int32)] * 4
                + [pltpu.VMEM((CH, d), jnp.float32)] * 2
                + [pltpu.VMEM_SHARED((N_PAD, d), jnp.float32)]
                + [pltpu.SemaphoreType.DMA] * 8
            ),
            name=f"gcn_prop_{tag}",
            compiler_params=pltpu.CompilerParams(use_tc_tiling_on_sc=False),
        )
        def prop(t_hbm, src_hbm, dst_hbm, zr_hbm, out_hbm, *refs):
            sbuf = refs[0:2]
            dbuf = refs[2:4]
            rbuf = refs[4:6]
            acc_sh = refs[6]
            sems = refs[7:]
            gsem = sems[0:2]
            ssem = sems[2:4]
            isem = sems[4:6]
            dsem = sems[6:8]
            c = lax.axis_index("c")
            s = lax.axis_index("s")
            wid = c * NS + s

            r0 = s * RPS
            for k in range(RPS // CHUNK):
                pltpu.sync_copy(zr_hbm, acc_sh.at[pl.ds(r0 + k * CHUNK,
                                                        CHUNK)])
            plsc.subcore_barrier()

            e0 = wid * EPW

            pltpu.sync_copy(src_hbm.at[pl.ds(e0, CH)], sbuf[0])
            pltpu.async_copy(src_hbm.at[pl.ds(e0 + CH, CH)], sbuf[1],
                             isem[1])
            pltpu.async_copy(dst_hbm.at[pl.ds(e0, CH)], dbuf[0], dsem[0])
            pltpu.async_copy(t_hbm.at[sbuf[0]], rbuf[0], gsem[0])

            def body(k, _):
                for p in range(2):
                    i = 2 * k + p
                    o = 1 - p
                    pltpu.make_async_copy(
                        t_hbm.at[sbuf[p]], rbuf[p], gsem[p]).wait()

                    @pl.when(i + 2 < NCH)
                    def _():
                        pltpu.async_copy(
                            src_hbm.at[pl.ds(e0 + (i + 2) * CH, CH)],
                            sbuf[p], isem[p])

                    pltpu.make_async_copy(
                        dst_hbm.at[pl.ds(e0 + i * CH, CH)],
                        dbuf[p], dsem[p]).wait()
                    pltpu.async_copy(rbuf[p], acc_sh.at[dbuf[p]],
                                     ssem[p], add=True)

                    @pl.when(jnp.logical_and(i >= 1, i + 1 < NCH))
                    def _():
                        pltpu.make_async_copy(
                            rbuf[o], acc_sh.at[dbuf[o]], ssem[o]).wait()

                    @pl.when(i + 1 < NCH)
                    def _():
                        j = i + 1
                        pltpu.make_async_copy(
                            src_hbm.at[pl.ds(e0 + j * CH, CH)],
                            sbuf[o], isem[o]).wait()
                        pltpu.async_copy(t_hbm.at[sbuf[o]], rbuf[o],
                                         gsem[o])
                        pltpu.async_copy(dst_hbm.at[pl.ds(e0 + j * CH, CH)],
                                         dbuf[o], dsem[o])
                return 0
            lax.fori_loop(0, NCH // 2, body, 0)

            for p in range(2):
                pltpu.make_async_copy(
                    rbuf[p], acc_sh.at[dbuf[p]], ssem[p]).wait()
            plsc.subcore_barrier()
            pltpu.sync_copy(acc_sh.at[pl.ds(r0, RPS)],
                            out_hbm.at[c, pl.ds(r0, RPS)])

        return prop

    return deg, make_prop(D1, "p1"), make_prop(D3, "p3")


def _tcA(deg3, x_p, w1p):
    def body(deg_ref, x_ref, w_ref, dis_ref, t_ref):
        deg = deg_ref[0] + deg_ref[1] + 1.0
        dis = lax.rsqrt(deg)
        dis_ref[...] = dis
        t_ref[...] = jnp.dot(x_ref[...], w_ref[...],
                             preferred_element_type=jnp.float32) * dis
    return pl.pallas_call(
        body,
        grid=(NBLK,),
        in_specs=[
            pl.BlockSpec((NC, BLK, 1), lambda i: (0, i, 0)),
            pl.BlockSpec((BLK, D_IN), lambda i: (i, 0)),
            pl.BlockSpec((D_IN, D1), lambda i: (0, 0)),
        ],
        out_specs=[
            pl.BlockSpec((BLK, 1), lambda i: (i, 0)),
            pl.BlockSpec((BLK, D1), lambda i: (i, 0)),
        ],
        out_shape=[
            jax.ShapeDtypeStruct((N_PAD, 1), jnp.float32),
            jax.ShapeDtypeStruct((N_PAD, D1), jnp.float32),
        ],
    )(deg3, x_p, w1p)


def _tcB(p1, t1, dis, b1p):
    def body(p_ref, t_ref, dis_ref, b_ref, u_ref):
        dis = dis_ref[...]
        h = dis * (p_ref[0] + p_ref[1] + t_ref[...]) + b_ref[...]
        u_ref[...] = jnp.maximum(h, 0.0) * dis
    return pl.pallas_call(
        body,
        grid=(NBLK,),
        in_specs=[
            pl.BlockSpec((NC, BLK, D1), lambda i: (0, i, 0)),
            pl.BlockSpec((BLK, D1), lambda i: (i, 0)),
            pl.BlockSpec((BLK, 1), lambda i: (i, 0)),
            pl.BlockSpec((1, D1), lambda i: (0, 0)),
        ],
        out_specs=pl.BlockSpec((BLK, D1), lambda i: (i, 0)),
        out_shape=jax.ShapeDtypeStruct((N_PAD, D1), jnp.float32),
    )(p1, t1, dis, b1p)


def _tcC(p2, u2, dis, w2p, b2p, w3p):
    def body(p_ref, u_ref, dis_ref, w2_ref, b2_ref, w3_ref, t3_ref):
        dis = dis_ref[...]
        ah = dis * (p_ref[0] + p_ref[1] + u_ref[...])
        h2 = jnp.maximum(
            jnp.dot(ah, w2_ref[...], preferred_element_type=jnp.float32)
            + b2_ref[...], 0.0)
        t3_ref[...] = jnp.dot(h2, w3_ref[...],
                              preferred_element_type=jnp.float32) * dis
    return pl.pallas_call(
        body,
        grid=(NBLK,),
        in_specs=[
            pl.BlockSpec((NC, BLK, D1), lambda i: (0, i, 0)),
            pl.BlockSpec((BLK, D1), lambda i: (i, 0)),
            pl.BlockSpec((BLK, 1), lambda i: (i, 0)),
            pl.BlockSpec((D1, D2), lambda i: (0, 0)),
            pl.BlockSpec((1, D2), lambda i: (0, 0)),
            pl.BlockSpec((D2, D3), lambda i: (0, 0)),
        ],
        out_specs=pl.BlockSpec((BLK, D3), lambda i: (i, 0)),
        out_shape=jax.ShapeDtypeStruct((N_PAD, D3), jnp.float32),
    )(p2, u2, dis, w2p, b2p, w3p)


def _tcD(batch_p, p3, t3, dis, b3p, bcol, wop, bo2):
    def body(batch_smem, p_ref, t_ref, dis_ref, b_ref, bcol_ref, wo_ref,
             bo_ref, pool_ref, out_ref):
        i = pl.program_id(0)

        @pl.when(i == 0)
        def _init():
            pool_ref[...] = jnp.full((NG, D3), -jnp.inf, jnp.float32)

        dis = dis_ref[...]
        h3 = jnp.maximum(
            dis * (p_ref[0] + p_ref[1] + t_ref[...]) + b_ref[...], 0.0)
        bcol = bcol_ref[...]
        g0 = batch_smem[i * BLK]
        g1 = jnp.minimum(batch_smem[i * BLK + BLK - 1], NG - 1)
        rowg = lax.broadcasted_iota(jnp.int32, (NG, D3), 0)

        def gbody(g, _):
            m = bcol == g
            v = jnp.where(m, h3, -jnp.inf)
            mx = jnp.max(v, axis=0, keepdims=True)
            upd = jnp.where(rowg == g, jnp.broadcast_to(mx, (NG, D3)),
                            -jnp.inf)
            pool_ref[...] = jnp.maximum(pool_ref[...], upd)
            return 0
        lax.fori_loop(g0, g1 + 1, gbody, 0)

        @pl.when(i == NBLK - 1)
        def _fin():
            z = jnp.dot(pool_ref[...], wo_ref[...],
                        preferred_element_type=jnp.float32) + bo_ref[...]
            out_ref[...] = jax.nn.sigmoid(z)

    pool, out = pl.pallas_call(
        body,
        grid=(NBLK,),
        in_specs=[
            pl.BlockSpec(memory_space=pltpu.SMEM),
            pl.BlockSpec((NC, BLK, D3), lambda i: (0, i, 0)),
            pl.BlockSpec((BLK, D3), lambda i: (i, 0)),
            pl.BlockSpec((BLK, 1), lambda i: (i, 0)),
            pl.BlockSpec((1, D3), lambda i: (0, 0)),
            pl.BlockSpec((BLK, 1), lambda i: (i, 0)),
            pl.BlockSpec((D3, 1), lambda i: (0, 0)),
            pl.BlockSpec((1, 1), lambda i: (0, 0)),
        ],
        out_specs=[
            pl.BlockSpec((NG, D3), lambda i: (0, 0)),
            pl.BlockSpec((NG, 1), lambda i: (0, 0)),
        ],
        out_shape=[
            jax.ShapeDtypeStruct((NG, D3), jnp.float32),
            jax.ShapeDtypeStruct((NG, 1), jnp.float32),
        ],
    )(batch_p, p3, t3, dis, b3p, bcol, wop, bo2)
    return out


def kernel(x, edge_index, batch, W1, b1, W2, b2, W3, b3, Wo, bo):
    src = edge_index[0]
    dst = edge_index[1]
    npad = E_PAD - E
    pad_ids = (N + (jnp.arange(npad, dtype=jnp.int32) % (N_PAD - N))
               ).astype(jnp.int32)
    src_p = jnp.concatenate([src, pad_ids])
    dst_p = jnp.concatenate([dst, pad_ids])

    x_p = jnp.pad(x, ((0, N_PAD - N), (0, 0)))
    w1p = jnp.pad(W1, ((0, 0), (0, D1 - 75)))
    b1p = jnp.pad(b1, (0, D1 - 75))[None, :]
    w2p = jnp.pad(W2, ((0, D1 - 75), (0, D2 - 150)))
    b2p = jnp.pad(b2, (0, D2 - 150))[None, :]
    w3p = jnp.pad(W3, ((0, D2 - 150), (0, D3 - 50)))
    b3p = jnp.pad(b3, (0, D3 - 50))[None, :]
    wop = jnp.pad(Wo, ((0, D3 - 50), (0, 0)))
    bo2 = bo[None, :]
    batch_p = jnp.pad(batch, (0, N_PAD - N), constant_values=NG)
    bcol = batch_p[:, None]

    _deg, _prop1, _prop3 = _sc_kernels()

    zr1 = jnp.zeros((CHUNK, D1), jnp.float32)
    zr3 = jnp.zeros((CHUNK, D3), jnp.float32)

    degp = _deg(dst_p)
    deg3 = degp[:, :, None]
    dis, t1 = _tcA(deg3, x_p, w1p)
    p1 = _prop1(t1, src_p, dst_p, zr1)
    u2 = _tcB(p1, t1, dis, b1p)
    p2 = _prop1(u2, src_p, dst_p, zr1)
    t3 = _tcC(p2, u2, dis, w2p, b2p, w3p)
    p3 = _prop3(t3, src_p, dst_p, zr3)
    return _tcD(batch_p, p3, t3, dis, b3p, bcol, wop, bo2)


# speedup vs baseline: 1.2493x; 1.1315x over previous
import functools

import jax
import jax.numpy as jnp
from jax import lax
from jax.experimental import pallas as pl
from jax.experimental.pallas import tpu as pltpu
from jax.experimental.pallas import tpu_sc as plsc

N = 10000
E = 320000
NG = 64
D_IN = 128

NC, NS = 2, 16
NW = NC * NS
N_PAD = 10240
RPS = N_PAD // NS
E_PAD = 327680
EPW = E_PAD // NW
CHUNK = 128
NCHUNK = EPW // CHUNK

BLK = 1024
NBLK = N_PAD // BLK

D1 = 80
D2 = 160
D3 = 64


@functools.cache
def _sc_kernels():
    mesh = plsc.VectorSubcoreMesh(
        core_axis_name="c", subcore_axis_name="s",
        num_cores=NC, num_subcores=NS)

    CDEG = 1024
    NCDEG = EPW // CDEG

    @functools.partial(
        pl.kernel,
        mesh=mesh,
        out_type=jax.ShapeDtypeStruct((NC, N_PAD), jnp.float32),
        scratch_types=[
            pltpu.VMEM((CDEG,), jnp.int32),
            pltpu.VMEM((CDEG,), jnp.int32),
            pltpu.VMEM((CDEG,), jnp.float32),
            pltpu.VMEM((RPS,), jnp.float32),
            pltpu.VMEM_SHARED((N_PAD,), jnp.float32),
            pltpu.SemaphoreType.DMA,
            pltpu.SemaphoreType.DMA,
            pltpu.SemaphoreType.DMA,
            pltpu.SemaphoreType.DMA,
        ],
        name="gcn_deg",
    )
    def deg(dst_hbm, out_hbm, dst_va, dst_vb, ones_v, zbuf_v, acc_sh,
            d0, d1, s0, s1):
        dbuf = (dst_va, dst_vb)
        dsem = (d0, d1)
        ssem = (s0, s1)
        c = lax.axis_index("c")
        s = lax.axis_index("s")
        wid = c * NS + s

        def fill(i, _):
            ones_v[pl.ds(i * 16, 16)] = jnp.full((16,), 1.0, jnp.float32)
            return 0
        lax.fori_loop(0, CDEG // 16, fill, 0)

        def zero(i, _):
            zbuf_v[pl.ds(i * 16, 16)] = jnp.zeros((16,), jnp.float32)
            return 0
        lax.fori_loop(0, RPS // 16, zero, 0)

        r0 = s * RPS
        pltpu.sync_copy(zbuf_v, acc_sh.at[pl.ds(r0, RPS)])
        plsc.subcore_barrier()

        e0 = wid * EPW
        for p in range(2):
            pltpu.async_copy(dst_hbm.at[pl.ds(e0 + p * CDEG, CDEG)],
                             dbuf[p], dsem[p])

        def body(k, _):
            for p in range(2):
                i = 2 * k + p
                pltpu.make_async_copy(
                    dst_hbm.at[pl.ds(e0 + i * CDEG, CDEG)],
                    dbuf[p], dsem[p]).wait()
                pltpu.async_copy(ones_v, acc_sh.at[dbuf[p]], ssem[p],
                                 add=True)

                @pl.when(i + 2 < NCDEG)
                def _():
                    pltpu.make_async_copy(
                        ones_v, acc_sh.at[dbuf[p]], ssem[p]).wait()
                    pltpu.async_copy(
                        dst_hbm.at[pl.ds(e0 + (i + 2) * CDEG, CDEG)],
                        dbuf[p], dsem[p])
            return 0
        lax.fori_loop(0, NCDEG // 2, body, 0)

        for p in range(2):
            pltpu.make_async_copy(
                ones_v, acc_sh.at[dbuf[p]], ssem[p]).wait()
        plsc.subcore_barrier()
        pltpu.sync_copy(acc_sh.at[pl.ds(r0, RPS)],
                        out_hbm.at[c, pl.ds(r0, RPS)])

    def make_prop(d, tag):
        CH = 320
        NCH = EPW // CH

        @functools.partial(
            pl.kernel,
            mesh=mesh,
            out_type=jax.ShapeDtypeStruct((NC, N_PAD, d), jnp.float32),
            scratch_types=(
                [pltpu.VMEM((CH,), jnp.int32)] * 4
                + [pltpu.VMEM((CH, d), jnp.float32)] * 2
                + [pltpu.VMEM_SHARED((N_PAD, d), jnp.float32)]
                + [pltpu.SemaphoreType.DMA] * 8
            ),
            name=f"gcn_prop_{tag}",
            compiler_params=pltpu.CompilerParams(use_tc_tiling_on_sc=False),
        )
        def prop(t_hbm, src_hbm, dst_hbm, zr_hbm, out_hbm, *refs):
            sbuf = refs[0:2]
            dbuf = refs[2:4]
            rbuf = refs[4:6]
            acc_sh = refs[6]
            sems = refs[7:]
            gsem = sems[0:2]
            ssem = sems[2:4]
            isem = sems[4:6]
            dsem = sems[6:8]
            c = lax.axis_index("c")
            s = lax.axis_index("s")
            wid = c * NS + s

            r0 = s * RPS
            for k in range(RPS // CHUNK):
                pltpu.sync_copy(zr_hbm, acc_sh.at[pl.ds(r0 + k * CHUNK,
                                                        CHUNK)])
            plsc.subcore_barrier()

            e0 = wid * EPW

            pltpu.sync_copy(src_hbm.at[pl.ds(e0, CH)], sbuf[0])
            pltpu.async_copy(src_hbm.at[pl.ds(e0 + CH, CH)], sbuf[1],
                             isem[1])
            pltpu.async_copy(dst_hbm.at[pl.ds(e0, CH)], dbuf[0], dsem[0])
            pltpu.async_copy(t_hbm.at[sbuf[0]], rbuf[0], gsem[0])

            def body(k, _):
                for p in range(2):
                    i = 2 * k + p
                    o = 1 - p
                    pltpu.make_async_copy(
                        t_hbm.at[sbuf[p]], rbuf[p], gsem[p]).wait()

                    @pl.when(i + 2 < NCH)
                    def _():
                        pltpu.async_copy(
                            src_hbm.at[pl.ds(e0 + (i + 2) * CH, CH)],
                            sbuf[p], isem[p])

                    pltpu.make_async_copy(
                        dst_hbm.at[pl.ds(e0 + i * CH, CH)],
                        dbuf[p], dsem[p]).wait()
                    pltpu.async_copy(rbuf[p], acc_sh.at[dbuf[p]],
                                     ssem[p], add=True)

                    @pl.when(jnp.logical_and(i >= 1, i + 1 < NCH))
                    def _():
                        pltpu.make_async_copy(
                            rbuf[o], acc_sh.at[dbuf[o]], ssem[o]).wait()

                    @pl.when(i + 1 < NCH)
                    def _():
                        j = i + 1
                        pltpu.make_async_copy(
                            src_hbm.at[pl.ds(e0 + j * CH, CH)],
                            sbuf[o], isem[o]).wait()
                        pltpu.async_copy(t_hbm.at[sbuf[o]], rbuf[o],
                                         gsem[o])
                        pltpu.async_copy(dst_hbm.at[pl.ds(e0 + j * CH, CH)],
                                         dbuf[o], dsem[o])
                return 0
            lax.fori_loop(0, NCH // 2, body, 0)

            for p in range(2):
                pltpu.make_async_copy(
                    rbuf[p], acc_sh.at[dbuf[p]], ssem[p]).wait()
            plsc.subcore_barrier()
            pltpu.sync_copy(acc_sh.at[pl.ds(r0, RPS)],
                            out_hbm.at[c, pl.ds(r0, RPS)])

        return prop

    return deg, make_prop(D1, "p1"), make_prop(D3, "p3")


def _tcA(deg3, x_p, w1p):
    def body(deg_ref, x_ref, w_ref, dis_ref, t_ref):
        deg = deg_ref[0] + deg_ref[1] + 1.0
        dis = lax.rsqrt(deg)
        dis_ref[...] = dis
        t_ref[...] = jnp.dot(x_ref[...], w_ref[...],
                             preferred_element_type=jnp.float32) * dis
    return pl.pallas_call(
        body,
        grid=(NBLK,),
        in_specs=[
            pl.BlockSpec((NC, BLK, 1), lambda i: (0, i, 0)),
            pl.BlockSpec((BLK, D_IN), lambda i: (i, 0)),
            pl.BlockSpec((D_IN, D1), lambda i: (0, 0)),
        ],
        out_specs=[
            pl.BlockSpec((BLK, 1), lambda i: (i, 0)),
            pl.BlockSpec((BLK, D1), lambda i: (i, 0)),
        ],
        out_shape=[
            jax.ShapeDtypeStruct((N_PAD, 1), jnp.float32),
            jax.ShapeDtypeStruct((N_PAD, D1), jnp.float32),
        ],
    )(deg3, x_p, w1p)


def _tcB(p1, t1, dis, b1p):
    def body(p_ref, t_ref, dis_ref, b_ref, u_ref):
        dis = dis_ref[...]
        h = dis * (p_ref[0] + p_ref[1] + t_ref[...]) + b_ref[...]
        u_ref[...] = jnp.maximum(h, 0.0) * dis
    return pl.pallas_call(
        body,
        grid=(NBLK,),
        in_specs=[
            pl.BlockSpec((NC, BLK, D1), lambda i: (0, i, 0)),
            pl.BlockSpec((BLK, D1), lambda i: (i, 0)),
            pl.BlockSpec((BLK, 1), lambda i: (i, 0)),
            pl.BlockSpec((1, D1), lambda i: (0, 0)),
        ],
        out_specs=pl.BlockSpec((BLK, D1), lambda i: (i, 0)),
        out_shape=jax.ShapeDtypeStruct((N_PAD, D1), jnp.float32),
    )(p1, t1, dis, b1p)


def _tcC(p2, u2, dis, w2p, b2p, w3p):
    def body(p_ref, u_ref, dis_ref, w2_ref, b2_ref, w3_ref, t3_ref):
        dis = dis_ref[...]
        ah = dis * (p_ref[0] + p_ref[1] + u_ref[...])
        h2 = jnp.maximum(
            jnp.dot(ah, w2_ref[...], preferred_element_type=jnp.float32)
            + b2_ref[...], 0.0)
        t3_ref[...] = jnp.dot(h2, w3_ref[...],
                              preferred_element_type=jnp.float32) * dis
    return pl.pallas_call(
        body,
        grid=(NBLK,),
        in_specs=[
            pl.BlockSpec((NC, BLK, D1), lambda i: (0, i, 0)),
            pl.BlockSpec((BLK, D1), lambda i: (i, 0)),
            pl.BlockSpec((BLK, 1), lambda i: (i, 0)),
            pl.BlockSpec((D1, D2), lambda i: (0, 0)),
            pl.BlockSpec((1, D2), lambda i: (0, 0)),
            pl.BlockSpec((D2, D3), lambda i: (0, 0)),
        ],
        out_specs=pl.BlockSpec((BLK, D3), lambda i: (i, 0)),
        out_shape=jax.ShapeDtypeStruct((N_PAD, D3), jnp.float32),
    )(p2, u2, dis, w2p, b2p, w3p)


def _tcD(batch_p, p3, t3, dis, b3p, bcol, wop, bo2):
    def body(batch_smem, p_ref, t_ref, dis_ref, b_ref, bcol_ref, wo_ref,
             bo_ref, pool_ref, out_ref):
        i = pl.program_id(0)

        @pl.when(i == 0)
        def _init():
            pool_ref[...] = jnp.full((NG, D3), -jnp.inf, jnp.float32)

        dis = dis_ref[...]
        h3 = jnp.maximum(
            dis * (p_ref[0] + p_ref[1] + t_ref[...]) + b_ref[...], 0.0)
        bcol = bcol_ref[...]
        g0 = batch_smem[i * BLK]
        g1 = jnp.minimum(batch_smem[i * BLK + BLK - 1], NG - 1)
        rowg = lax.broadcasted_iota(jnp.int32, (NG, D3), 0)

        def gbody(g, _):
            m = bcol == g
            v = jnp.where(m, h3, -jnp.inf)
            mx = jnp.max(v, axis=0, keepdims=True)
            upd = jnp.where(rowg == g, jnp.broadcast_to(mx, (NG, D3)),
                            -jnp.inf)
            pool_ref[...] = jnp.maximum(pool_ref[...], upd)
            return 0
        lax.fori_loop(g0, g1 + 1, gbody, 0)

        @pl.when(i == NBLK - 1)
        def _fin():
            z = jnp.dot(pool_ref[...], wo_ref[...],
                        preferred_element_type=jnp.float32) + bo_ref[...]
            out_ref[...] = jax.nn.sigmoid(z)

    pool, out = pl.pallas_call(
        body,
        grid=(NBLK,),
        in_specs=[
            pl.BlockSpec(memory_space=pltpu.SMEM),
            pl.BlockSpec((NC, BLK, D3), lambda i: (0, i, 0)),
            pl.BlockSpec((BLK, D3), lambda i: (i, 0)),
            pl.BlockSpec((BLK, 1), lambda i: (i, 0)),
            pl.BlockSpec((1, D3), lambda i: (0, 0)),
            pl.BlockSpec((BLK, 1), lambda i: (i, 0)),
            pl.BlockSpec((D3, 1), lambda i: (0, 0)),
            pl.BlockSpec((1, 1), lambda i: (0, 0)),
        ],
        out_specs=[
            pl.BlockSpec((NG, D3), lambda i: (0, 0)),
            pl.BlockSpec((NG, 1), lambda i: (0, 0)),
        ],
        out_shape=[
            jax.ShapeDtypeStruct((NG, D3), jnp.float32),
            jax.ShapeDtypeStruct((NG, 1), jnp.float32),
        ],
    )(batch_p, p3, t3, dis, b3p, bcol, wop, bo2)
    return out


def kernel(x, edge_index, batch, W1, b1, W2, b2, W3, b3, Wo, bo):
    src = edge_index[0]
    dst = edge_index[1]
    npad = E_PAD - E
    pad_ids = (N + (jnp.arange(npad, dtype=jnp.int32) % (N_PAD - N))
               ).astype(jnp.int32)
    src_p = jnp.concatenate([src, pad_ids])
    dst_p = jnp.concatenate([dst, pad_ids])

    x_p = jnp.pad(x, ((0, N_PAD - N), (0, 0)))
    w1p = jnp.pad(W1, ((0, 0), (0, D1 - 75)))
    b1p = jnp.pad(b1, (0, D1 - 75))[None, :]
    w2p = jnp.pad(W2, ((0, D1 - 75), (0, D2 - 150)))
    b2p = jnp.pad(b2, (0, D2 - 150))[None, :]
    w3p = jnp.pad(W3, ((0, D2 - 150), (0, D3 - 50)))
    b3p = jnp.pad(b3, (0, D3 - 50))[None, :]
    wop = jnp.pad(Wo, ((0, D3 - 50), (0, 0)))
    bo2 = bo[None, :]
    batch_p = jnp.pad(batch, (0, N_PAD - N), constant_values=NG)
    bcol = batch_p[:, None]

    _deg, _prop1, _prop3 = _sc_kernels()

    zr1 = jnp.zeros((CHUNK, D1), jnp.float32)
    zr3 = jnp.zeros((CHUNK, D3), jnp.float32)

    degp = _deg(dst_p)
    deg3 = degp[:, :, None]
    dis, t1 = _tcA(deg3, x_p, w1p)
    p1 = _prop1(t1, src_p, dst_p, zr1)
    u2 = _tcB(p1, t1, dis, b1p)
    p2 = _prop1(u2, src_p, dst_p, zr1)
    t3 = _tcC(p2, u2, dis, w2p, b2p, w3p)
    p3 = _prop3(t3, src_p, dst_p, zr3)
    return _tcD(batch_p, p3, t3, dis, b3p, bcol, wop, bo2)


# speedup vs baseline: 1.2556x; 1.0050x over previous
import functools

import jax
import jax.numpy as jnp
from jax import lax
from jax.experimental import pallas as pl
from jax.experimental.pallas import tpu as pltpu
from jax.experimental.pallas import tpu_sc as plsc

N = 10000
E = 320000
NG = 64
D_IN = 128

NC, NS = 2, 16
NW = NC * NS
N_PAD = 10240
RPS = N_PAD // NS
E_PAD = 327680
EPW = E_PAD // NW
CHUNK = 128
NCHUNK = EPW // CHUNK

BLK = 1024
NBLK = N_PAD // BLK

D1 = 80
D2 = 160
D3 = 64


@functools.cache
def _sc_kernels():
    mesh = plsc.VectorSubcoreMesh(
        core_axis_name="c", subcore_axis_name="s",
        num_cores=NC, num_subcores=NS)

    CDEG = 1024
    NCDEG = EPW // CDEG

    @functools.partial(
        pl.kernel,
        mesh=mesh,
        out_type=jax.ShapeDtypeStruct((NC, N_PAD), jnp.float32),
        scratch_types=[
            pltpu.VMEM((CDEG,), jnp.int32),
            pltpu.VMEM((CDEG,), jnp.int32),
            pltpu.VMEM((CDEG,), jnp.float32),
            pltpu.VMEM((RPS,), jnp.float32),
            pltpu.VMEM_SHARED((N_PAD,), jnp.float32),
            pltpu.SemaphoreType.DMA,
            pltpu.SemaphoreType.DMA,
            pltpu.SemaphoreType.DMA,
            pltpu.SemaphoreType.DMA,
        ],
        name="gcn_deg",
    )
    def deg(dst_hbm, out_hbm, dst_va, dst_vb, ones_v, zbuf_v, acc_sh,
            d0, d1, s0, s1):
        dbuf = (dst_va, dst_vb)
        dsem = (d0, d1)
        ssem = (s0, s1)
        c = lax.axis_index("c")
        s = lax.axis_index("s")
        wid = c * NS + s

        def fill(i, _):
            ones_v[pl.ds(i * 16, 16)] = jnp.full((16,), 1.0, jnp.float32)
            return 0
        lax.fori_loop(0, CDEG // 16, fill, 0)

        def zero(i, _):
            zbuf_v[pl.ds(i * 16, 16)] = jnp.zeros((16,), jnp.float32)
            return 0
        lax.fori_loop(0, RPS // 16, zero, 0)

        r0 = s * RPS
        pltpu.sync_copy(zbuf_v, acc_sh.at[pl.ds(r0, RPS)])
        plsc.subcore_barrier()

        e0 = wid * EPW
        for p in range(2):
            pltpu.async_copy(dst_hbm.at[pl.ds(e0 + p * CDEG, CDEG)],
                             dbuf[p], dsem[p])

        def body(k, _):
            for p in range(2):
                i = 2 * k + p
                pltpu.make_async_copy(
                    dst_hbm.at[pl.ds(e0 + i * CDEG, CDEG)],
                    dbuf[p], dsem[p]).wait()
                pltpu.async_copy(ones_v, acc_sh.at[dbuf[p]], ssem[p],
                                 add=True)

                @pl.when(i + 2 < NCDEG)
                def _():
                    pltpu.make_async_copy(
                        ones_v, acc_sh.at[dbuf[p]], ssem[p]).wait()
                    pltpu.async_copy(
                        dst_hbm.at[pl.ds(e0 + (i + 2) * CDEG, CDEG)],
                        dbuf[p], dsem[p])
            return 0
        lax.fori_loop(0, NCDEG // 2, body, 0)

        for p in range(2):
            pltpu.make_async_copy(
                ones_v, acc_sh.at[dbuf[p]], ssem[p]).wait()
        plsc.subcore_barrier()
        pltpu.sync_copy(acc_sh.at[pl.ds(r0, RPS)],
                        out_hbm.at[c, pl.ds(r0, RPS)])

    def make_prop(d, tag, ch):
        CH = ch
        NCH = EPW // CH

        @functools.partial(
            pl.kernel,
            mesh=mesh,
            out_type=jax.ShapeDtypeStruct((NC, N_PAD, d), jnp.float32),
            scratch_types=(
                [pltpu.VMEM((CH,), jnp.int32)] * 4
                + [pltpu.VMEM((CH, d), jnp.float32)] * 2
                + [pltpu.VMEM_SHARED((N_PAD, d), jnp.float32)]
                + [pltpu.SemaphoreType.DMA] * 8
            ),
            name=f"gcn_prop_{tag}",
            compiler_params=pltpu.CompilerParams(use_tc_tiling_on_sc=False),
        )
        def prop(t_hbm, src_hbm, dst_hbm, zr_hbm, out_hbm, *refs):
            sbuf = refs[0:2]
            dbuf = refs[2:4]
            rbuf = refs[4:6]
            acc_sh = refs[6]
            sems = refs[7:]
            gsem = sems[0:2]
            ssem = sems[2:4]
            isem = sems[4:6]
            dsem = sems[6:8]
            c = lax.axis_index("c")
            s = lax.axis_index("s")
            wid = c * NS + s

            r0 = s * RPS
            for k in range(RPS // CHUNK):
                pltpu.sync_copy(zr_hbm, acc_sh.at[pl.ds(r0 + k * CHUNK,
                                                        CHUNK)])
            plsc.subcore_barrier()

            e0 = wid * EPW

            pltpu.sync_copy(src_hbm.at[pl.ds(e0, CH)], sbuf[0])
            pltpu.async_copy(src_hbm.at[pl.ds(e0 + CH, CH)], sbuf[1],
                             isem[1])
            pltpu.async_copy(dst_hbm.at[pl.ds(e0, CH)], dbuf[0], dsem[0])
            pltpu.async_copy(t_hbm.at[sbuf[0]], rbuf[0], gsem[0])

            def body(k, _):
                for p in range(2):
                    i = 2 * k + p
                    o = 1 - p
                    pltpu.make_async_copy(
                        t_hbm.at[sbuf[p]], rbuf[p], gsem[p]).wait()

                    @pl.when(i + 2 < NCH)
                    def _():
                        pltpu.async_copy(
                            src_hbm.at[pl.ds(e0 + (i + 2) * CH, CH)],
                            sbuf[p], isem[p])

                    pltpu.make_async_copy(
                        dst_hbm.at[pl.ds(e0 + i * CH, CH)],
                        dbuf[p], dsem[p]).wait()
                    pltpu.async_copy(rbuf[p], acc_sh.at[dbuf[p]],
                                     ssem[p], add=True)

                    @pl.when(jnp.logical_and(i >= 1, i + 1 < NCH))
                    def _():
                        pltpu.make_async_copy(
                            rbuf[o], acc_sh.at[dbuf[o]], ssem[o]).wait()

                    @pl.when(i + 1 < NCH)
                    def _():
                        j = i + 1
                        pltpu.make_async_copy(
                            src_hbm.at[pl.ds(e0 + j * CH, CH)],
                            sbuf[o], isem[o]).wait()
                        pltpu.async_copy(t_hbm.at[sbuf[o]], rbuf[o],
                                         gsem[o])
                        pltpu.async_copy(dst_hbm.at[pl.ds(e0 + j * CH, CH)],
                                         dbuf[o], dsem[o])
                return 0
            lax.fori_loop(0, NCH // 2, body, 0)

            for p in range(2):
                pltpu.make_async_copy(
                    rbuf[p], acc_sh.at[dbuf[p]], ssem[p]).wait()
            plsc.subcore_barrier()
            pltpu.sync_copy(acc_sh.at[pl.ds(r0, RPS)],
                            out_hbm.at[c, pl.ds(r0, RPS)])

        return prop

    return deg, make_prop(D1, "p1", 320), make_prop(D3, "p3", 512)


def _tcA(deg3, x_p, w1p):
    def body(deg_ref, x_ref, w_ref, dis_ref, t_ref):
        deg = deg_ref[0] + deg_ref[1] + 1.0
        dis = lax.rsqrt(deg)
        dis_ref[...] = dis
        t_ref[...] = jnp.dot(x_ref[...], w_ref[...],
                             preferred_element_type=jnp.float32) * dis
    return pl.pallas_call(
        body,
        grid=(NBLK,),
        in_specs=[
            pl.BlockSpec((NC, BLK, 1), lambda i: (0, i, 0)),
            pl.BlockSpec((BLK, D_IN), lambda i: (i, 0)),
            pl.BlockSpec((D_IN, D1), lambda i: (0, 0)),
        ],
        out_specs=[
            pl.BlockSpec((BLK, 1), lambda i: (i, 0)),
            pl.BlockSpec((BLK, D1), lambda i: (i, 0)),
        ],
        out_shape=[
            jax.ShapeDtypeStruct((N_PAD, 1), jnp.float32),
            jax.ShapeDtypeStruct((N_PAD, D1), jnp.float32),
        ],
    )(deg3, x_p, w1p)


def _tcB(p1, t1, dis, b1p):
    def body(p_ref, t_ref, dis_ref, b_ref, u_ref):
        dis = dis_ref[...]
        h = dis * (p_ref[0] + p_ref[1] + t_ref[...]) + b_ref[...]
        u_ref[...] = jnp.maximum(h, 0.0) * dis
    return pl.pallas_call(
        body,
        grid=(NBLK,),
        in_specs=[
            pl.BlockSpec((NC, BLK, D1), lambda i: (0, i, 0)),
            pl.BlockSpec((BLK, D1), lambda i: (i, 0)),
            pl.BlockSpec((BLK, 1), lambda i: (i, 0)),
            pl.BlockSpec((1, D1), lambda i: (0, 0)),
        ],
        out_specs=pl.BlockSpec((BLK, D1), lambda i: (i, 0)),
        out_shape=jax.ShapeDtypeStruct((N_PAD, D1), jnp.float32),
    )(p1, t1, dis, b1p)


def _tcC(p2, u2, dis, w2p, b2p, w3p):
    def body(p_ref, u_ref, dis_ref, w2_ref, b2_ref, w3_ref, t3_ref):
        dis = dis_ref[...]
        ah = dis * (p_ref[0] + p_ref[1] + u_ref[...])
        h2 = jnp.maximum(
            jnp.dot(ah, w2_ref[...], preferred_element_type=jnp.float32)
            + b2_ref[...], 0.0)
        t3_ref[...] = jnp.dot(h2, w3_ref[...],
                              preferred_element_type=jnp.float32) * dis
    return pl.pallas_call(
        body,
        grid=(NBLK,),
        in_specs=[
            pl.BlockSpec((NC, BLK, D1), lambda i: (0, i, 0)),
            pl.BlockSpec((BLK, D1), lambda i: (i, 0)),
            pl.BlockSpec((BLK, 1), lambda i: (i, 0)),
            pl.BlockSpec((D1, D2), lambda i: (0, 0)),
            pl.BlockSpec((1, D2), lambda i: (0, 0)),
            pl.BlockSpec((D2, D3), lambda i: (0, 0)),
        ],
        out_specs=pl.BlockSpec((BLK, D3), lambda i: (i, 0)),
        out_shape=jax.ShapeDtypeStruct((N_PAD, D3), jnp.float32),
    )(p2, u2, dis, w2p, b2p, w3p)


def _tcD(batch_p, p3, t3, dis, b3p, bcol, wop, bo2):
    def body(batch_smem, p_ref, t_ref, dis_ref, b_ref, bcol_ref, wo_ref,
             bo_ref, pool_ref, out_ref):
        i = pl.program_id(0)

        @pl.when(i == 0)
        def _init():
            pool_ref[...] = jnp.full((NG, D3), -jnp.inf, jnp.float32)

        dis = dis_ref[...]
        h3 = jnp.maximum(
            dis * (p_ref[0] + p_ref[1] + t_ref[...]) + b_ref[...], 0.0)
        bcol = bcol_ref[...]
        g0 = batch_smem[i * BLK]
        g1 = jnp.minimum(batch_smem[i * BLK + BLK - 1], NG - 1)
        rowg = lax.broadcasted_iota(jnp.int32, (NG, D3), 0)

        def gbody(g, _):
            m = bcol == g
            v = jnp.where(m, h3, -jnp.inf)
            mx = jnp.max(v, axis=0, keepdims=True)
            upd = jnp.where(rowg == g, jnp.broadcast_to(mx, (NG, D3)),
                            -jnp.inf)
            pool_ref[...] = jnp.maximum(pool_ref[...], upd)
            return 0
        lax.fori_loop(g0, g1 + 1, gbody, 0)

        @pl.when(i == NBLK - 1)
        def _fin():
            z = jnp.dot(pool_ref[...], wo_ref[...],
                        preferred_element_type=jnp.float32) + bo_ref[...]
            out_ref[...] = jax.nn.sigmoid(z)

    pool, out = pl.pallas_call(
        body,
        grid=(NBLK,),
        in_specs=[
            pl.BlockSpec(memory_space=pltpu.SMEM),
            pl.BlockSpec((NC, BLK, D3), lambda i: (0, i, 0)),
            pl.BlockSpec((BLK, D3), lambda i: (i, 0)),
            pl.BlockSpec((BLK, 1), lambda i: (i, 0)),
            pl.BlockSpec((1, D3), lambda i: (0, 0)),
            pl.BlockSpec((BLK, 1), lambda i: (i, 0)),
            pl.BlockSpec((D3, 1), lambda i: (0, 0)),
            pl.BlockSpec((1, 1), lambda i: (0, 0)),
        ],
        out_specs=[
            pl.BlockSpec((NG, D3), lambda i: (0, 0)),
            pl.BlockSpec((NG, 1), lambda i: (0, 0)),
        ],
        out_shape=[
            jax.ShapeDtypeStruct((NG, D3), jnp.float32),
            jax.ShapeDtypeStruct((NG, 1), jnp.float32),
        ],
    )(batch_p, p3, t3, dis, b3p, bcol, wop, bo2)
    return out


def kernel(x, edge_index, batch, W1, b1, W2, b2, W3, b3, Wo, bo):
    src = edge_index[0]
    dst = edge_index[1]
    npad = E_PAD - E
    pad_ids = (N + (jnp.arange(npad, dtype=jnp.int32) % (N_PAD - N))
               ).astype(jnp.int32)
    src_p = jnp.concatenate([src, pad_ids])
    dst_p = jnp.concatenate([dst, pad_ids])

    x_p = jnp.pad(x, ((0, N_PAD - N), (0, 0)))
    w1p = jnp.pad(W1, ((0, 0), (0, D1 - 75)))
    b1p = jnp.pad(b1, (0, D1 - 75))[None, :]
    w2p = jnp.pad(W2, ((0, D1 - 75), (0, D2 - 150)))
    b2p = jnp.pad(b2, (0, D2 - 150))[None, :]
    w3p = jnp.pad(W3, ((0, D2 - 150), (0, D3 - 50)))
    b3p = jnp.pad(b3, (0, D3 - 50))[None, :]
    wop = jnp.pad(Wo, ((0, D3 - 50), (0, 0)))
    bo2 = bo[None, :]
    batch_p = jnp.pad(batch, (0, N_PAD - N), constant_values=NG)
    bcol = batch_p[:, None]

    _deg, _prop1, _prop3 = _sc_kernels()

    zr1 = jnp.zeros((CHUNK, D1), jnp.float32)
    zr3 = jnp.zeros((CHUNK, D3), jnp.float32)

    degp = _deg(dst_p)
    deg3 = degp[:, :, None]
    dis, t1 = _tcA(deg3, x_p, w1p)
    p1 = _prop1(t1, src_p, dst_p, zr1)
    u2 = _tcB(p1, t1, dis, b1p)
    p2 = _prop1(u2, src_p, dst_p, zr1)
    t3 = _tcC(p2, u2, dis, w2p, b2p, w3p)
    p3 = _prop3(t3, src_p, dst_p, zr3)
    return _tcD(batch_p, p3, t3, dis, b3p, bcol, wop, bo2)


# speedup vs baseline: 1.2817x; 1.0208x over previous
import functools

import jax
import jax.numpy as jnp
from jax import lax
from jax.experimental import pallas as pl
from jax.experimental.pallas import tpu as pltpu
from jax.experimental.pallas import tpu_sc as plsc

N = 10000
E = 320000
NG = 64
D_IN = 128

NC, NS = 2, 16
NW = NC * NS
N_PAD = 10240
RPS = N_PAD // NS
E_PAD = 327680
EPW = E_PAD // NW
CHUNK = 128
NCHUNK = EPW // CHUNK

BLK = 1024
NBLK = N_PAD // BLK

D1 = 80
D2 = 160
D3 = 64


@functools.cache
def _sc_kernels():
    mesh = plsc.VectorSubcoreMesh(
        core_axis_name="c", subcore_axis_name="s",
        num_cores=NC, num_subcores=NS)

    CDEG = 1024
    NCDEG = EPW // CDEG

    @functools.partial(
        pl.kernel,
        mesh=mesh,
        out_type=jax.ShapeDtypeStruct((NC, N_PAD), jnp.float32),
        scratch_types=[
            pltpu.VMEM((CDEG,), jnp.int32),
            pltpu.VMEM((CDEG,), jnp.int32),
            pltpu.VMEM((CDEG,), jnp.float32),
            pltpu.VMEM((RPS,), jnp.float32),
            pltpu.VMEM_SHARED((N_PAD,), jnp.float32),
            pltpu.SemaphoreType.DMA,
            pltpu.SemaphoreType.DMA,
            pltpu.SemaphoreType.DMA,
            pltpu.SemaphoreType.DMA,
        ],
        name="gcn_deg",
    )
    def deg(dst_hbm, out_hbm, dst_va, dst_vb, ones_v, zbuf_v, acc_sh,
            d0, d1, s0, s1):
        dbuf = (dst_va, dst_vb)
        dsem = (d0, d1)
        ssem = (s0, s1)
        c = lax.axis_index("c")
        s = lax.axis_index("s")
        wid = c * NS + s

        def fill(i, _):
            ones_v[pl.ds(i * 16, 16)] = jnp.full((16,), 1.0, jnp.float32)
            return 0
        lax.fori_loop(0, CDEG // 16, fill, 0)

        def zero(i, _):
            zbuf_v[pl.ds(i * 16, 16)] = jnp.zeros((16,), jnp.float32)
            return 0
        lax.fori_loop(0, RPS // 16, zero, 0)

        r0 = s * RPS
        pltpu.sync_copy(zbuf_v, acc_sh.at[pl.ds(r0, RPS)])
        plsc.subcore_barrier()

        e0 = wid * EPW
        for p in range(2):
            pltpu.async_copy(dst_hbm.at[pl.ds(e0 + p * CDEG, CDEG)],
                             dbuf[p], dsem[p])

        def body(k, _):
            for p in range(2):
                i = 2 * k + p
                pltpu.make_async_copy(
                    dst_hbm.at[pl.ds(e0 + i * CDEG, CDEG)],
                    dbuf[p], dsem[p]).wait()
                pltpu.async_copy(ones_v, acc_sh.at[dbuf[p]], ssem[p],
                                 add=True)

                @pl.when(i + 2 < NCDEG)
                def _():
                    pltpu.make_async_copy(
                        ones_v, acc_sh.at[dbuf[p]], ssem[p]).wait()
                    pltpu.async_copy(
                        dst_hbm.at[pl.ds(e0 + (i + 2) * CDEG, CDEG)],
                        dbuf[p], dsem[p])
            return 0
        lax.fori_loop(0, NCDEG // 2, body, 0)

        for p in range(2):
            pltpu.make_async_copy(
                ones_v, acc_sh.at[dbuf[p]], ssem[p]).wait()
        plsc.subcore_barrier()
        pltpu.sync_copy(acc_sh.at[pl.ds(r0, RPS)],
                        out_hbm.at[c, pl.ds(r0, RPS)])

    def make_prop(d, tag, ch):
        CH = ch
        NCH = EPW // CH

        @functools.partial(
            pl.kernel,
            mesh=mesh,
            out_type=jax.ShapeDtypeStruct((NC, N_PAD, d), jnp.float32),
            scratch_types=(
                [pltpu.VMEM((CH,), jnp.int32)] * 4
                + [pltpu.VMEM((CH, d), jnp.float32)] * 2
                + [pltpu.VMEM_SHARED((N_PAD, d), jnp.float32)]
                + [pltpu.SemaphoreType.DMA] * 8
            ),
            name=f"gcn_prop_{tag}",
            compiler_params=pltpu.CompilerParams(use_tc_tiling_on_sc=False),
        )
        def prop(t_hbm, src_hbm, dst_hbm, zr_hbm, out_hbm, *refs):
            sbuf = refs[0:2]
            dbuf = refs[2:4]
            rbuf = refs[4:6]
            acc_sh = refs[6]
            sems = refs[7:]
            gsem = sems[0:2]
            ssem = sems[2:4]
            isem = sems[4:6]
            dsem = sems[6:8]
            c = lax.axis_index("c")
            s = lax.axis_index("s")
            wid = c * NS + s

            r0 = s * RPS
            pltpu.sync_copy(zr_hbm, acc_sh.at[pl.ds(r0, RPS)])
            plsc.subcore_barrier()

            e0 = wid * EPW

            pltpu.sync_copy(src_hbm.at[pl.ds(e0, CH)], sbuf[0])
            pltpu.async_copy(src_hbm.at[pl.ds(e0 + CH, CH)], sbuf[1],
                             isem[1])
            pltpu.async_copy(dst_hbm.at[pl.ds(e0, CH)], dbuf[0], dsem[0])
            pltpu.async_copy(t_hbm.at[sbuf[0]], rbuf[0], gsem[0])

            def body(k, _):
                for p in range(2):
                    i = 2 * k + p
                    o = 1 - p
                    pltpu.make_async_copy(
                        t_hbm.at[sbuf[p]], rbuf[p], gsem[p]).wait()

                    @pl.when(i + 2 < NCH)
                    def _():
                        pltpu.async_copy(
                            src_hbm.at[pl.ds(e0 + (i + 2) * CH, CH)],
                            sbuf[p], isem[p])

                    pltpu.make_async_copy(
                        dst_hbm.at[pl.ds(e0 + i * CH, CH)],
                        dbuf[p], dsem[p]).wait()
                    pltpu.async_copy(rbuf[p], acc_sh.at[dbuf[p]],
                                     ssem[p], add=True)

                    @pl.when(jnp.logical_and(i >= 1, i + 1 < NCH))
                    def _():
                        pltpu.make_async_copy(
                            rbuf[o], acc_sh.at[dbuf[o]], ssem[o]).wait()

                    @pl.when(i + 1 < NCH)
                    def _():
                        j = i + 1
                        pltpu.make_async_copy(
                            src_hbm.at[pl.ds(e0 + j * CH, CH)],
                            sbuf[o], isem[o]).wait()
                        pltpu.async_copy(t_hbm.at[sbuf[o]], rbuf[o],
                                         gsem[o])
                        pltpu.async_copy(dst_hbm.at[pl.ds(e0 + j * CH, CH)],
                                         dbuf[o], dsem[o])
                return 0
            lax.fori_loop(0, NCH // 2, body, 0)

            for p in range(2):
                pltpu.make_async_copy(
                    rbuf[p], acc_sh.at[dbuf[p]], ssem[p]).wait()
            plsc.subcore_barrier()
            pltpu.sync_copy(acc_sh.at[pl.ds(r0, RPS)],
                            out_hbm.at[c, pl.ds(r0, RPS)])

        return prop

    return deg, make_prop(D1, "p1", 320), make_prop(D3, "p3", 512)


def _tcA1(x_p, w1p):
    def body(x_ref, w_ref, t_ref):
        t_ref[...] = jnp.dot(x_ref[...], w_ref[...],
                             preferred_element_type=jnp.float32)
    return pl.pallas_call(
        body,
        grid=(NBLK,),
        in_specs=[
            pl.BlockSpec((BLK, D_IN), lambda i: (i, 0)),
            pl.BlockSpec((D_IN, D1), lambda i: (0, 0)),
        ],
        out_specs=pl.BlockSpec((BLK, D1), lambda i: (i, 0)),
        out_shape=jax.ShapeDtypeStruct((N_PAD, D1), jnp.float32),
    )(x_p, w1p)


def _tcA2(deg3, t1r):
    def body(deg_ref, t_ref, dis_ref, out_ref):
        deg = deg_ref[0] + deg_ref[1] + 1.0
        dis = lax.rsqrt(deg)
        dis_ref[...] = dis
        out_ref[...] = t_ref[...] * dis
    return pl.pallas_call(
        body,
        grid=(NBLK,),
        in_specs=[
            pl.BlockSpec((NC, BLK, 1), lambda i: (0, i, 0)),
            pl.BlockSpec((BLK, D1), lambda i: (i, 0)),
        ],
        out_specs=[
            pl.BlockSpec((BLK, 1), lambda i: (i, 0)),
            pl.BlockSpec((BLK, D1), lambda i: (i, 0)),
        ],
        out_shape=[
            jax.ShapeDtypeStruct((N_PAD, 1), jnp.float32),
            jax.ShapeDtypeStruct((N_PAD, D1), jnp.float32),
        ],
    )(deg3, t1r)


def _tcB(p1, t1, dis, b1p):
    def body(p_ref, t_ref, dis_ref, b_ref, u_ref):
        dis = dis_ref[...]
        h = dis * (p_ref[0] + p_ref[1] + t_ref[...]) + b_ref[...]
        u_ref[...] = jnp.maximum(h, 0.0) * dis
    return pl.pallas_call(
        body,
        grid=(NBLK,),
        in_specs=[
            pl.BlockSpec((NC, BLK, D1), lambda i: (0, i, 0)),
            pl.BlockSpec((BLK, D1), lambda i: (i, 0)),
            pl.BlockSpec((BLK, 1), lambda i: (i, 0)),
            pl.BlockSpec((1, D1), lambda i: (0, 0)),
        ],
        out_specs=pl.BlockSpec((BLK, D1), lambda i: (i, 0)),
        out_shape=jax.ShapeDtypeStruct((N_PAD, D1), jnp.float32),
    )(p1, t1, dis, b1p)


def _tcC(p2, u2, dis, w2p, b2p, w3p):
    def body(p_ref, u_ref, dis_ref, w2_ref, b2_ref, w3_ref, t3_ref):
        dis = dis_ref[...]
        ah = dis * (p_ref[0] + p_ref[1] + u_ref[...])
        h2 = jnp.maximum(
            jnp.dot(ah, w2_ref[...], preferred_element_type=jnp.float32)
            + b2_ref[...], 0.0)
        t3_ref[...] = jnp.dot(h2, w3_ref[...],
                              preferred_element_type=jnp.float32) * dis
    return pl.pallas_call(
        body,
        grid=(NBLK,),
        in_specs=[
            pl.BlockSpec((NC, BLK, D1), lambda i: (0, i, 0)),
            pl.BlockSpec((BLK, D1), lambda i: (i, 0)),
            pl.BlockSpec((BLK, 1), lambda i: (i, 0)),
            pl.BlockSpec((D1, D2), lambda i: (0, 0)),
            pl.BlockSpec((1, D2), lambda i: (0, 0)),
            pl.BlockSpec((D2, D3), lambda i: (0, 0)),
        ],
        out_specs=pl.BlockSpec((BLK, D3), lambda i: (i, 0)),
        out_shape=jax.ShapeDtypeStruct((N_PAD, D3), jnp.float32),
    )(p2, u2, dis, w2p, b2p, w3p)


def _tcD(batch_p, p3, t3, dis, b3p, bcol, wop, bo2):
    def body(batch_smem, p_ref, t_ref, dis_ref, b_ref, bcol_ref, wo_ref,
             bo_ref, pool_ref, out_ref):
        i = pl.program_id(0)

        @pl.when(i == 0)
        def _init():
            pool_ref[...] = jnp.full((NG, D3), -jnp.inf, jnp.float32)

        dis = dis_ref[...]
        h3 = jnp.maximum(
            dis * (p_ref[0] + p_ref[1] + t_ref[...]) + b_ref[...], 0.0)
        bcol = bcol_ref[...]
        g0 = batch_smem[i * BLK]
        g1 = jnp.minimum(batch_smem[i * BLK + BLK - 1], NG - 1)
        rowg = lax.broadcasted_iota(jnp.int32, (NG, D3), 0)

        def gbody(g, _):
            m = bcol == g
            v = jnp.where(m, h3, -jnp.inf)
            mx = jnp.max(v, axis=0, keepdims=True)
            upd = jnp.where(rowg == g, jnp.broadcast_to(mx, (NG, D3)),
                            -jnp.inf)
            pool_ref[...] = jnp.maximum(pool_ref[...], upd)
            return 0
        lax.fori_loop(g0, g1 + 1, gbody, 0)

        @pl.when(i == NBLK - 1)
        def _fin():
            z = jnp.dot(pool_ref[...], wo_ref[...],
                        preferred_element_type=jnp.float32) + bo_ref[...]
            out_ref[...] = jax.nn.sigmoid(z)

    pool, out = pl.pallas_call(
        body,
        grid=(NBLK,),
        in_specs=[
            pl.BlockSpec(memory_space=pltpu.SMEM),
            pl.BlockSpec((NC, BLK, D3), lambda i: (0, i, 0)),
            pl.BlockSpec((BLK, D3), lambda i: (i, 0)),
            pl.BlockSpec((BLK, 1), lambda i: (i, 0)),
            pl.BlockSpec((1, D3), lambda i: (0, 0)),
            pl.BlockSpec((BLK, 1), lambda i: (i, 0)),
            pl.BlockSpec((D3, 1), lambda i: (0, 0)),
            pl.BlockSpec((1, 1), lambda i: (0, 0)),
        ],
        out_specs=[
            pl.BlockSpec((NG, D3), lambda i: (0, 0)),
            pl.BlockSpec((NG, 1), lambda i: (0, 0)),
        ],
        out_shape=[
            jax.ShapeDtypeStruct((NG, D3), jnp.float32),
            jax.ShapeDtypeStruct((NG, 1), jnp.float32),
        ],
    )(batch_p, p3, t3, dis, b3p, bcol, wop, bo2)
    return out


def kernel(x, edge_index, batch, W1, b1, W2, b2, W3, b3, Wo, bo):
    src = edge_index[0]
    dst = edge_index[1]
    npad = E_PAD - E
    pad_ids = (N + (jnp.arange(npad, dtype=jnp.int32) % (N_PAD - N))
               ).astype(jnp.int32)
    src_p = jnp.concatenate([src, pad_ids])
    dst_p = jnp.concatenate([dst, pad_ids])

    x_p = jnp.pad(x, ((0, N_PAD - N), (0, 0)))
    w1p = jnp.pad(W1, ((0, 0), (0, D1 - 75)))
    b1p = jnp.pad(b1, (0, D1 - 75))[None, :]
    w2p = jnp.pad(W2, ((0, D1 - 75), (0, D2 - 150)))
    b2p = jnp.pad(b2, (0, D2 - 150))[None, :]
    w3p = jnp.pad(W3, ((0, D2 - 150), (0, D3 - 50)))
    b3p = jnp.pad(b3, (0, D3 - 50))[None, :]
    wop = jnp.pad(Wo, ((0, D3 - 50), (0, 0)))
    bo2 = bo[None, :]
    batch_p = jnp.pad(batch, (0, N_PAD - N), constant_values=NG)
    bcol = batch_p[:, None]

    _deg, _prop1, _prop3 = _sc_kernels()

    zr1 = jnp.zeros((RPS, D1), jnp.float32)
    zr3 = jnp.zeros((RPS, D3), jnp.float32)

    t1r = _tcA1(x_p, w1p)
    degp = _deg(dst_p)
    deg3 = degp[:, :, None]
    dis, t1 = _tcA2(deg3, t1r)
    p1 = _prop1(t1, src_p, dst_p, zr1)
    u2 = _tcB(p1, t1, dis, b1p)
    p2 = _prop1(u2, src_p, dst_p, zr1)
    t3 = _tcC(p2, u2, dis, w2p, b2p, w3p)
    p3 = _prop3(t3, src_p, dst_p, zr3)
    return _tcD(batch_p, p3, t3, dis, b3p, bcol, wop, bo2)


# speedup vs baseline: 1.2842x; 1.0020x over previous
import functools

import jax
import jax.numpy as jnp
from jax import lax
from jax.experimental import pallas as pl
from jax.experimental.pallas import tpu as pltpu
from jax.experimental.pallas import tpu_sc as plsc

N = 10000
E = 320000
NG = 64
D_IN = 128

NC, NS = 2, 16
NW = NC * NS
N_PAD = 10240
RPS = N_PAD // NS
E_PAD = 327680
EPW = E_PAD // NW
CHUNK = 128
NCHUNK = EPW // CHUNK

BLK = 1024
NBLK = N_PAD // BLK

D1 = 80
D2 = 160
D3 = 64


@functools.cache
def _sc_kernels():
    mesh = plsc.VectorSubcoreMesh(
        core_axis_name="c", subcore_axis_name="s",
        num_cores=NC, num_subcores=NS)

    CDEG = 1024
    NCDEG = EPW // CDEG

    @functools.partial(
        pl.kernel,
        mesh=mesh,
        out_type=jax.ShapeDtypeStruct((NC, N_PAD), jnp.float32),
        scratch_types=[
            pltpu.VMEM((CDEG,), jnp.int32),
            pltpu.VMEM((CDEG,), jnp.int32),
            pltpu.VMEM((CDEG,), jnp.float32),
            pltpu.VMEM((RPS,), jnp.float32),
            pltpu.VMEM_SHARED((N_PAD,), jnp.float32),
            pltpu.SemaphoreType.DMA,
            pltpu.SemaphoreType.DMA,
            pltpu.SemaphoreType.DMA,
            pltpu.SemaphoreType.DMA,
        ],
        name="gcn_deg",
    )
    def deg(dst_hbm, out_hbm, dst_va, dst_vb, ones_v, zbuf_v, acc_sh,
            d0, d1, s0, s1):
        dbuf = (dst_va, dst_vb)
        dsem = (d0, d1)
        ssem = (s0, s1)
        c = lax.axis_index("c")
        s = lax.axis_index("s")
        wid = c * NS + s

        def fill(i, _):
            ones_v[pl.ds(i * 16, 16)] = jnp.full((16,), 1.0, jnp.float32)
            return 0
        lax.fori_loop(0, CDEG // 16, fill, 0)

        def zero(i, _):
            zbuf_v[pl.ds(i * 16, 16)] = jnp.zeros((16,), jnp.float32)
            return 0
        lax.fori_loop(0, RPS // 16, zero, 0)

        r0 = s * RPS
        pltpu.sync_copy(zbuf_v, acc_sh.at[pl.ds(r0, RPS)])
        plsc.subcore_barrier()

        e0 = wid * EPW
        for p in range(2):
            pltpu.async_copy(dst_hbm.at[pl.ds(e0 + p * CDEG, CDEG)],
                             dbuf[p], dsem[p])

        def body(k, _):
            for p in range(2):
                i = 2 * k + p
                pltpu.make_async_copy(
                    dst_hbm.at[pl.ds(e0 + i * CDEG, CDEG)],
                    dbuf[p], dsem[p]).wait()
                pltpu.async_copy(ones_v, acc_sh.at[dbuf[p]], ssem[p],
                                 add=True)

                @pl.when(i + 2 < NCDEG)
                def _():
                    pltpu.make_async_copy(
                        ones_v, acc_sh.at[dbuf[p]], ssem[p]).wait()
                    pltpu.async_copy(
                        dst_hbm.at[pl.ds(e0 + (i + 2) * CDEG, CDEG)],
                        dbuf[p], dsem[p])
            return 0
        lax.fori_loop(0, NCDEG // 2, body, 0)

        for p in range(2):
            pltpu.make_async_copy(
                ones_v, acc_sh.at[dbuf[p]], ssem[p]).wait()
        plsc.subcore_barrier()
        pltpu.sync_copy(acc_sh.at[pl.ds(r0, RPS)],
                        out_hbm.at[c, pl.ds(r0, RPS)])

    def make_prop(d, tag, ch):
        CH = ch
        NCH = EPW // CH

        @functools.partial(
            pl.kernel,
            mesh=mesh,
            out_type=jax.ShapeDtypeStruct((NC, N_PAD, d), jnp.float32),
            scratch_types=(
                [pltpu.VMEM((CH,), jnp.int32)] * 4
                + [pltpu.VMEM((CH, d), jnp.float32)] * 2
                + [pltpu.VMEM_SHARED((N_PAD, d), jnp.float32)]
                + [pltpu.SemaphoreType.DMA] * 8
            ),
            name=f"gcn_prop_{tag}",
            compiler_params=pltpu.CompilerParams(use_tc_tiling_on_sc=False),
        )
        def prop(t_hbm, src_hbm, dst_hbm, zr_hbm, out_hbm, *refs):
            sbuf = refs[0:2]
            dbuf = refs[2:4]
            rbuf = refs[4:6]
            acc_sh = refs[6]
            sems = refs[7:]
            gsem = sems[0:2]
            ssem = sems[2:4]
            isem = sems[4:6]
            dsem = sems[6:8]
            c = lax.axis_index("c")
            s = lax.axis_index("s")
            wid = c * NS + s

            r0 = s * RPS
            pltpu.sync_copy(zr_hbm, acc_sh.at[pl.ds(r0, RPS)])
            plsc.subcore_barrier()

            e0 = wid * EPW

            pltpu.sync_copy(src_hbm.at[pl.ds(e0, CH)], sbuf[0])
            pltpu.async_copy(src_hbm.at[pl.ds(e0 + CH, CH)], sbuf[1],
                             isem[1])
            pltpu.async_copy(dst_hbm.at[pl.ds(e0, CH)], dbuf[0], dsem[0])
            pltpu.async_copy(t_hbm.at[sbuf[0]], rbuf[0], gsem[0])

            def body(k, _):
                for p in range(2):
                    i = 2 * k + p
                    o = 1 - p
                    pltpu.make_async_copy(
                        t_hbm.at[sbuf[p]], rbuf[p], gsem[p]).wait()

                    @pl.when(i + 2 < NCH)
                    def _():
                        pltpu.async_copy(
                            src_hbm.at[pl.ds(e0 + (i + 2) * CH, CH)],
                            sbuf[p], isem[p])

                    pltpu.make_async_copy(
                        dst_hbm.at[pl.ds(e0 + i * CH, CH)],
                        dbuf[p], dsem[p]).wait()
                    pltpu.async_copy(rbuf[p], acc_sh.at[dbuf[p]],
                                     ssem[p], add=True)

                    @pl.when(jnp.logical_and(i >= 1, i + 1 < NCH))
                    def _():
                        pltpu.make_async_copy(
                            rbuf[o], acc_sh.at[dbuf[o]], ssem[o]).wait()

                    @pl.when(i + 1 < NCH)
                    def _():
                        j = i + 1
                        pltpu.make_async_copy(
                            src_hbm.at[pl.ds(e0 + j * CH, CH)],
                            sbuf[o], isem[o]).wait()
                        pltpu.async_copy(t_hbm.at[sbuf[o]], rbuf[o],
                                         gsem[o])
                        pltpu.async_copy(dst_hbm.at[pl.ds(e0 + j * CH, CH)],
                                         dbuf[o], dsem[o])
                return 0
            lax.fori_loop(0, NCH // 2, body, 0)

            for p in range(2):
                pltpu.make_async_copy(
                    rbuf[p], acc_sh.at[dbuf[p]], ssem[p]).wait()
            plsc.subcore_barrier()
            pltpu.sync_copy(acc_sh.at[pl.ds(r0, RPS)],
                            out_hbm.at[c, pl.ds(r0, RPS)])

        return prop

    return deg, make_prop(D1, "p1", 320), make_prop(D3, "p3", 640)


def _tcA1(x_p, w1p):
    def body(x_ref, w_ref, t_ref):
        t_ref[...] = jnp.dot(x_ref[...], w_ref[...],
                             preferred_element_type=jnp.float32)
    return pl.pallas_call(
        body,
        grid=(NBLK,),
        in_specs=[
            pl.BlockSpec((BLK, D_IN), lambda i: (i, 0)),
            pl.BlockSpec((D_IN, D1), lambda i: (0, 0)),
        ],
        out_specs=pl.BlockSpec((BLK, D1), lambda i: (i, 0)),
        out_shape=jax.ShapeDtypeStruct((N_PAD, D1), jnp.float32),
    )(x_p, w1p)


def _tcA2(deg3, t1r):
    def body(deg_ref, t_ref, dis_ref, out_ref):
        deg = deg_ref[0] + deg_ref[1] + 1.0
        dis = lax.rsqrt(deg)
        dis_ref[...] = dis
        out_ref[...] = t_ref[...] * dis
    return pl.pallas_call(
        body,
        grid=(NBLK,),
        in_specs=[
            pl.BlockSpec((NC, BLK, 1), lambda i: (0, i, 0)),
            pl.BlockSpec((BLK, D1), lambda i: (i, 0)),
        ],
        out_specs=[
            pl.BlockSpec((BLK, 1), lambda i: (i, 0)),
            pl.BlockSpec((BLK, D1), lambda i: (i, 0)),
        ],
        out_shape=[
            jax.ShapeDtypeStruct((N_PAD, 1), jnp.float32),
            jax.ShapeDtypeStruct((N_PAD, D1), jnp.float32),
        ],
    )(deg3, t1r)


def _tcB(p1, t1, dis, b1p):
    def body(p_ref, t_ref, dis_ref, b_ref, u_ref):
        dis = dis_ref[...]
        h = dis * (p_ref[0] + p_ref[1] + t_ref[...]) + b_ref[...]
        u_ref[...] = jnp.maximum(h, 0.0) * dis
    return pl.pallas_call(
        body,
        grid=(NBLK,),
        in_specs=[
            pl.BlockSpec((NC, BLK, D1), lambda i: (0, i, 0)),
            pl.BlockSpec((BLK, D1), lambda i: (i, 0)),
            pl.BlockSpec((BLK, 1), lambda i: (i, 0)),
            pl.BlockSpec((1, D1), lambda i: (0, 0)),
        ],
        out_specs=pl.BlockSpec((BLK, D1), lambda i: (i, 0)),
        out_shape=jax.ShapeDtypeStruct((N_PAD, D1), jnp.float32),
    )(p1, t1, dis, b1p)


def _tcC(p2, u2, dis, w2p, b2p, w3p):
    def body(p_ref, u_ref, dis_ref, w2_ref, b2_ref, w3_ref, t3_ref):
        dis = dis_ref[...]
        ah = dis * (p_ref[0] + p_ref[1] + u_ref[...])
        h2 = jnp.maximum(
            jnp.dot(ah, w2_ref[...], preferred_element_type=jnp.float32)
            + b2_ref[...], 0.0)
        t3_ref[...] = jnp.dot(h2, w3_ref[...],
                              preferred_element_type=jnp.float32) * dis
    return pl.pallas_call(
        body,
        grid=(NBLK,),
        in_specs=[
            pl.BlockSpec((NC, BLK, D1), lambda i: (0, i, 0)),
            pl.BlockSpec((BLK, D1), lambda i: (i, 0)),
            pl.BlockSpec((BLK, 1), lambda i: (i, 0)),
            pl.BlockSpec((D1, D2), lambda i: (0, 0)),
            pl.BlockSpec((1, D2), lambda i: (0, 0)),
            pl.BlockSpec((D2, D3), lambda i: (0, 0)),
        ],
        out_specs=pl.BlockSpec((BLK, D3), lambda i: (i, 0)),
        out_shape=jax.ShapeDtypeStruct((N_PAD, D3), jnp.float32),
    )(p2, u2, dis, w2p, b2p, w3p)


def _tcD(batch_p, p3, t3, dis, b3p, bcol, wop, bo2):
    def body(batch_smem, p_ref, t_ref, dis_ref, b_ref, bcol_ref, wo_ref,
             bo_ref, pool_ref, out_ref):
        i = pl.program_id(0)

        @pl.when(i == 0)
        def _init():
            pool_ref[...] = jnp.full((NG, D3), -jnp.inf, jnp.float32)

        dis = dis_ref[...]
        h3 = jnp.maximum(
            dis * (p_ref[0] + p_ref[1] + t_ref[...]) + b_ref[...], 0.0)
        bcol = bcol_ref[...]
        g0 = batch_smem[i * BLK]
        g1 = jnp.minimum(batch_smem[i * BLK + BLK - 1], NG - 1)
        rowg = lax.broadcasted_iota(jnp.int32, (NG, D3), 0)

        def gbody(g, _):
            m = bcol == g
            v = jnp.where(m, h3, -jnp.inf)
            mx = jnp.max(v, axis=0, keepdims=True)
            upd = jnp.where(rowg == g, jnp.broadcast_to(mx, (NG, D3)),
                            -jnp.inf)
            pool_ref[...] = jnp.maximum(pool_ref[...], upd)
            return 0
        lax.fori_loop(g0, g1 + 1, gbody, 0)

        @pl.when(i == NBLK - 1)
        def _fin():
            z = jnp.dot(pool_ref[...], wo_ref[...],
                        preferred_element_type=jnp.float32) + bo_ref[...]
            out_ref[...] = jax.nn.sigmoid(z)

    pool, out = pl.pallas_call(
        body,
        grid=(NBLK,),
        in_specs=[
            pl.BlockSpec(memory_space=pltpu.SMEM),
            pl.BlockSpec((NC, BLK, D3), lambda i: (0, i, 0)),
            pl.BlockSpec((BLK, D3), lambda i: (i, 0)),
            pl.BlockSpec((BLK, 1), lambda i: (i, 0)),
            pl.BlockSpec((1, D3), lambda i: (0, 0)),
            pl.BlockSpec((BLK, 1), lambda i: (i, 0)),
            pl.BlockSpec((D3, 1), lambda i: (0, 0)),
            pl.BlockSpec((1, 1), lambda i: (0, 0)),
        ],
        out_specs=[
            pl.BlockSpec((NG, D3), lambda i: (0, 0)),
            pl.BlockSpec((NG, 1), lambda i: (0, 0)),
        ],
        out_shape=[
            jax.ShapeDtypeStruct((NG, D3), jnp.float32),
            jax.ShapeDtypeStruct((NG, 1), jnp.float32),
        ],
    )(batch_p, p3, t3, dis, b3p, bcol, wop, bo2)
    return out


def kernel(x, edge_index, batch, W1, b1, W2, b2, W3, b3, Wo, bo):
    src = edge_index[0]
    dst = edge_index[1]
    npad = E_PAD - E
    pad_ids = (N + (jnp.arange(npad, dtype=jnp.int32) % (N_PAD - N))
               ).astype(jnp.int32)
    src_p = jnp.concatenate([src, pad_ids])
    dst_p = jnp.concatenate([dst, pad_ids])

    x_p = jnp.pad(x, ((0, N_PAD - N), (0, 0)))
    w1p = jnp.pad(W1, ((0, 0), (0, D1 - 75)))
    b1p = jnp.pad(b1, (0, D1 - 75))[None, :]
    w2p = jnp.pad(W2, ((0, D1 - 75), (0, D2 - 150)))
    b2p = jnp.pad(b2, (0, D2 - 150))[None, :]
    w3p = jnp.pad(W3, ((0, D2 - 150), (0, D3 - 50)))
    b3p = jnp.pad(b3, (0, D3 - 50))[None, :]
    wop = jnp.pad(Wo, ((0, D3 - 50), (0, 0)))
    bo2 = bo[None, :]
    batch_p = jnp.pad(batch, (0, N_PAD - N), constant_values=NG)
    bcol = batch_p[:, None]

    _deg, _prop1, _prop3 = _sc_kernels()

    zr1 = jnp.zeros((RPS, D1), jnp.float32)
    zr3 = jnp.zeros((RPS, D3), jnp.float32)

    t1r = _tcA1(x_p, w1p)
    degp = _deg(dst_p)
    deg3 = degp[:, :, None]
    dis, t1 = _tcA2(deg3, t1r)
    p1 = _prop1(t1, src_p, dst_p, zr1)
    u2 = _tcB(p1, t1, dis, b1p)
    p2 = _prop1(u2, src_p, dst_p, zr1)
    t3 = _tcC(p2, u2, dis, w2p, b2p, w3p)
    p3 = _prop3(t3, src_p, dst_p, zr3)
    return _tcD(batch_p, p3, t3, dis, b3p, bcol, wop, bo2)
